# baseline (speedup 1.0000x reference)
import jax
import jax.numpy as jnp
from jax.experimental import pallas as pl
from jax.experimental.pallas import tpu as pltpu
from jax.experimental.pallas import tpu_sc as plsc

B, N, D, VOCAB = 8, 2048, 16, 30
GW = 128


def _sc_gather(emb, tok_flat):
    @pl.kernel(out_type=jax.ShapeDtypeStruct((B * N, 128), jnp.float32),
               mesh=plsc.VectorSubcoreMesh(core_axis_name="c",
                                           subcore_axis_name="s"))
    def k(x_hbm, i_hbm, o_hbm):
        def body(i_vmem, o_vmem):
            pltpu.sync_copy(x_hbm.at[i_vmem.at[0]], o_vmem)

        pltpu.emit_pipeline(
            body,
            grid=(B * N // GW,),
            in_specs=[pl.BlockSpec((1, GW), index_map=lambda i: (0, i))],
            out_specs=[pl.BlockSpec((GW, 128), index_map=lambda i: (i, 0))],
            core_axis_name="s",
            dimension_semantics=(pltpu.PARALLEL,),
        )(i_hbm, o_hbm)

    return k(emb, tok_flat)


def _gcn_body(adj_ref, h0_ref, w1_ref, b1_ref, w2_ref, b2_ref,
              w3_ref, b3_ref, out_ref):
    A = adj_ref[0]
    t = jnp.dot(h0_ref[0, :, :D], w1_ref[...], preferred_element_type=jnp.float32)
    H = N // 2
    sums = []
    for w_ref, b_ref in ((None, b1_ref), (w2_ref, b2_ref), (w3_ref, b3_ref)):
        g0 = jnp.dot(A[:H], t, preferred_element_type=jnp.float32)
        g1 = jnp.dot(A[H:], t, preferred_element_type=jnp.float32)
        g = jnp.concatenate([g0, g1], axis=0)
        if w_ref is not None:
            g = jnp.dot(g, w_ref[...], preferred_element_type=jnp.float32)
        t = jnp.maximum(g + b_ref[...], 0.0)
        sums.append(jnp.sum(t, axis=0, keepdims=True))
    out_ref[0] = jnp.concatenate(sums, axis=0)


def kernel(adj, tokens, emb_table, W1, b1, W2, b2, W3, b3):
    tok_flat = tokens.astype(jnp.int32).reshape(1, B * N)
    emb128 = jnp.pad(emb_table, ((0, 0), (0, 128 - D)))
    h0 = _sc_gather(emb128, tok_flat).reshape(B, N, 128)
    full = lambda s: pl.BlockSpec(s, lambda b: tuple(0 for _ in s))
    out = pl.pallas_call(
        _gcn_body,
        grid=(B,),
        in_specs=[
            pl.BlockSpec((1, N, N), lambda b: (b, 0, 0)),
            pl.BlockSpec((1, N, 128), lambda b: (b, 0, 0)),
            full((D, D)), full((1, D)),
            full((D, D)), full((1, D)),
            full((D, D)), full((1, D)),
        ],
        out_specs=pl.BlockSpec((1, 3, D), lambda b: (b, 0, 0)),
        out_shape=jax.ShapeDtypeStruct((B, 3, D), jnp.float32),
    )(adj, h0,
      W1, b1.reshape(1, D), W2, b2.reshape(1, D), W3, b3.reshape(1, D))
    return out.reshape(B, 3 * D)


# speedup vs baseline: 2.2342x; 2.2342x over previous
import jax
import jax.numpy as jnp
from jax.experimental import pallas as pl

B, N, D, VOCAB = 8, 2048, 16, 30


def _gcn_body(adj_ref, tok_ref, emb_ref, w1_ref, b1_ref, w2_ref, b2_ref,
              w3_ref, b3_ref, out_ref):
    A = adj_ref[0]
    ot = (tok_ref[0] == jax.lax.broadcasted_iota(jnp.int32, (VOCAB, N), 0))
    ew1 = jnp.dot(emb_ref[...], w1_ref[...], preferred_element_type=jnp.float32)
    t = jax.lax.dot_general(ot.astype(jnp.float32), ew1,
                            (((0,), (0,)), ((), ())),
                            preferred_element_type=jnp.float32)
    H = N // 2
    sums = []
    for w_ref, b_ref in ((None, b1_ref), (w2_ref, b2_ref), (w3_ref, b3_ref)):
        g0 = jnp.dot(A[:H], t, preferred_element_type=jnp.float32)
        g1 = jnp.dot(A[H:], t, preferred_element_type=jnp.float32)
        g = jnp.concatenate([g0, g1], axis=0)
        if w_ref is not None:
            g = jnp.dot(g, w_ref[...], preferred_element_type=jnp.float32)
        t = jnp.maximum(g + b_ref[...], 0.0)
        sums.append(jnp.sum(t, axis=0, keepdims=True))
    out_ref[0] = jnp.concatenate(sums, axis=0)


def kernel(adj, tokens, emb_table, W1, b1, W2, b2, W3, b3):
    tok = tokens.astype(jnp.int32).reshape(B, 1, N)
    full = lambda s: pl.BlockSpec(s, lambda b: tuple(0 for _ in s))
    out = pl.pallas_call(
        _gcn_body,
        grid=(B,),
        in_specs=[
            pl.BlockSpec((1, N, N), lambda b: (b, 0, 0)),
            pl.BlockSpec((1, 1, N), lambda b: (b, 0, 0)),
            full((VOCAB, D)),
            full((D, D)), full((1, D)),
            full((D, D)), full((1, D)),
            full((D, D)), full((1, D)),
        ],
        out_specs=pl.BlockSpec((1, 3, D), lambda b: (b, 0, 0)),
        out_shape=jax.ShapeDtypeStruct((B, 3, D), jnp.float32),
    )(adj, tok, emb_table,
      W1, b1.reshape(1, D), W2, b2.reshape(1, D), W3, b3.reshape(1, D))
    return out.reshape(B, 3 * D)
